# baseline (speedup 1.0000x reference)
import jax
import jax.numpy as jnp
from jax import lax
from jax.experimental import pallas as pl
from jax.experimental.pallas import tpu as pltpu
from jax.experimental.pallas import tpu_sc as plsc

B, L, D = 1024, 200, 128
N = B * L
EPS = 1e-12

NC, NS, LANES = 2, 16, 16
NW = NC * NS
ROWS_PER_W = N // NW
CHUNK = 64
NCHUNK = ROWS_PER_W // CHUNK
NPAIR = NCHUNK // 2
SEG = D // LANES
GRP = CHUNK // LANES
WPOS_ROWS = 200
PPOS_OFF = 200


def _rsqrt(x):
    i = lax.bitcast_convert_type(x, jnp.int32)
    i = jnp.int32(0x5F3759DF) - lax.shift_right_arithmetic(i, 1)
    y = lax.bitcast_convert_type(i, jnp.float32)
    for _ in range(3):
        y = y * (1.5 - 0.5 * x * y * y)
    return y


def _body(ids_hbm, wtab, wpos, ppos, grid, cate, gamma_hbm, beta_hbm,
          out_hbm, ids_v, bufs0, bufs1, wp_v, gb_v,
          gsem0, gsem1, osem0, osem1, isem):
    wid = lax.axis_index("s") * NC + lax.axis_index("c")
    base = wid * ROWS_PER_W

    bufs = (bufs0, bufs1)
    gsem = (gsem0, gsem1)
    osem = (osem0, osem1)
    tabs = ((0, wtab), (3, grid), (4, cate))

    pltpu.sync_copy(wpos, wp_v.at[pl.ds(0, WPOS_ROWS)])
    pltpu.sync_copy(ppos, wp_v.at[pl.ds(PPOS_OFF, 50)])
    pltpu.sync_copy(gamma_hbm, gb_v.at[0])
    pltpu.sync_copy(beta_hbm, gb_v.at[1])
    gvec = [gb_v[0, pl.ds(s * LANES, LANES)] for s in range(SEG)]
    bvec = [gb_v[1, pl.ds(s * LANES, LANES)] for s in range(SEG)]
    cols = [lax.iota(jnp.int32, LANES) + (s * LANES) for s in range(SEG)]

    def idx_copy(p, slot):
        return pltpu.make_async_copy(
            ids_hbm.at[:, pl.ds(base + p * (2 * CHUNK), 2 * CHUNK)],
            ids_v.at[slot, pl.ds(0, 5)], isem)

    def issue(slot, off, s):
        for k, (j, tab) in enumerate(tabs):
            pltpu.async_copy(
                tab.at[ids_v.at[slot, j, pl.ds(off, CHUNK)]],
                bufs[s].at[k], gsem[s])

    def drain_gathers(slot, off, s):
        for k, (j, tab) in enumerate(tabs):
            pltpu.make_async_copy(
                tab.at[ids_v.at[slot, j, pl.ds(off, CHUNK)]],
                bufs[s].at[k], gsem[s]).wait()

    def out_copy(g, s):
        return pltpu.make_async_copy(
            bufs[s].at[0], out_hbm.at[pl.ds(base + g * CHUNK, CHUNK)],
            osem[s])

    idx_copy(0, 0).start()
    idx_copy(0, 0).wait()
    issue(0, 0, 0)

    def quad_body(q, carry):
        for u in (0, 1):
          p = 2 * q + u
          pa = u
          pb = 1 - u
          @pl.when(p + 1 < NPAIR)
          def _():
              idx_copy(p + 1, pb).start()

          for s in (0, 1):
            g = 2 * p + s
            nxt = 1 - s
            @pl.when(g + 1 < NCHUNK)
            def _():
                @pl.when(g >= 1)
                def _():
                    out_copy(g - 1, nxt).wait()
                if s == 0:
                    issue(pa, CHUNK, nxt)
                else:
                    idx_copy(p + 1, pb).wait()
                    issue(pb, 0, nxt)

            drain_gathers(pa, s * CHUNK, s)
            b = bufs[s]
            off = s * CHUNK

            def group(gi, rc):
                lane0 = jnp.full((LANES,), pa, jnp.int32)
                lanec = cols[0] + (off + gi * LANES)
                widx = plsc.load_gather(
                    ids_v, [lane0, jnp.full((LANES,), 1, jnp.int32), lanec])
                pidx = plsc.load_gather(
                    ids_v, [lane0, jnp.full((LANES,), 2, jnp.int32), lanec])
                for r in range(LANES):
                    row = gi * LANES + r
                    wrow = lax.broadcast(widx[r], (LANES,))
                    prow = lax.broadcast(pidx[r] + PPOS_OFF, (LANES,))
                    vs = []
                    for seg in range(SEG):
                        sl = pl.ds(seg * LANES, LANES)
                        wv = plsc.load_gather(wp_v, [wrow, cols[seg]])
                        pv = plsc.load_gather(wp_v, [prow, cols[seg]])
                        vs.append(b[0, row, sl] + b[1, row, sl]
                                  + b[2, row, sl] + wv + pv)
                    tot = vs[0]
                    for seg in range(1, SEG):
                        tot = tot + vs[seg]
                    mean = lax.broadcast(jnp.sum(tot) * (1.0 / D), (LANES,))
                    xs = [v - mean for v in vs]
                    sq = xs[0] * xs[0]
                    for seg in range(1, SEG):
                        sq = sq + xs[seg] * xs[seg]
                    var = lax.broadcast(
                        jnp.sum(sq) * (1.0 / D) + EPS, (LANES,))
                    rstd = _rsqrt(var)
                    for seg in range(SEG):
                        sl = pl.ds(seg * LANES, LANES)
                        b[0, row, sl] = xs[seg] * rstd * gvec[seg] + bvec[seg]
                return rc

            lax.fori_loop(0, GRP, group, 0, unroll=1)
            out_copy(g, s).start()
        return carry

    lax.fori_loop(0, NPAIR // 2, quad_body, 0, unroll=1)
    out_copy(NCHUNK - 2, 0).wait()
    out_copy(NCHUNK - 1, 1).wait()


@jax.jit
def _poiembed_sc(ids, word_tab, wpos_tab, ppos_tab, grid_tab, cate_tab,
                 ln_gamma, ln_beta):
    mesh = plsc.VectorSubcoreMesh(core_axis_name="c", subcore_axis_name="s")
    f = pl.kernel(
        _body,
        out_type=jax.ShapeDtypeStruct((N, D), jnp.float32),
        mesh=mesh,
        compiler_params=pltpu.CompilerParams(needs_layout_passes=False),
        scratch_types=[
            pltpu.VMEM((2, 8, 2 * CHUNK), jnp.int32),
            pltpu.VMEM((3, CHUNK, D), jnp.float32),
            pltpu.VMEM((3, CHUNK, D), jnp.float32),
            pltpu.VMEM((WPOS_ROWS + 56, D), jnp.float32),
            pltpu.VMEM((2, D), jnp.float32),
            pltpu.SemaphoreType.DMA,
            pltpu.SemaphoreType.DMA,
            pltpu.SemaphoreType.DMA,
            pltpu.SemaphoreType.DMA,
            pltpu.SemaphoreType.DMA,
        ],
    )
    return f(ids, word_tab, wpos_tab, ppos_tab, grid_tab, cate_tab,
             ln_gamma, ln_beta)


def kernel(poi_name_token_ids, word_level_pos_ids, poi_level_pos_ids,
           grid_level_pos_ids, poi_cate_ids,
           word_tab, wpos_tab, ppos_tab, grid_tab, cate_tab,
           ln_gamma, ln_beta):
    ids = jnp.stack([
        poi_name_token_ids.reshape(-1),
        word_level_pos_ids.reshape(-1),
        poi_level_pos_ids.reshape(-1),
        grid_level_pos_ids.reshape(-1),
        poi_cate_ids.reshape(-1),
    ], axis=0)
    out = _poiembed_sc(ids, word_tab, wpos_tab, ppos_tab, grid_tab,
                       cate_tab, ln_gamma, ln_beta)
    return out.reshape(B, L, D)


# speedup vs baseline: 1.7348x; 1.7348x over previous
import jax
import jax.numpy as jnp
from jax import lax
from jax.experimental import pallas as pl
from jax.experimental.pallas import tpu as pltpu
from jax.experimental.pallas import tpu_sc as plsc

B, L, D = 1024, 200, 128
N = B * L
EPS = 1e-12

NC, NS, LANES = 2, 16, 16
NW = NC * NS
ROWS_PER_W = N // NW
CHUNK = 64
NCHUNK = ROWS_PER_W // CHUNK
NPAIR = NCHUNK // 2
SEG = D // LANES
SPLIT = 2
SUB = CHUNK // SPLIT


def _rsqrt(x):
    i = lax.bitcast_convert_type(x, jnp.int32)
    i = jnp.int32(0x5F3759DF) - lax.shift_right_arithmetic(i, 1)
    y = lax.bitcast_convert_type(i, jnp.float32)
    for _ in range(3):
        y = y * (1.5 - 0.5 * x * y * y)
    return y


def _body(ids_hbm, wtab, wpos, ppos, grid, cate, gamma_hbm, beta_hbm,
          out_hbm, ids_v, bufs0, bufs1, gb_v, gsem0, gsem1, osem0, osem1,
          isem):
    wid = lax.axis_index("s") * NC + lax.axis_index("c")
    base = wid * ROWS_PER_W

    bufs = (bufs0, bufs1)
    gsem = (gsem0, gsem1)
    osem = (osem0, osem1)
    tabs = (wtab, wpos, ppos, grid, cate)

    pltpu.sync_copy(gamma_hbm, gb_v.at[0])
    pltpu.sync_copy(beta_hbm, gb_v.at[1])
    gvec = [gb_v[0, pl.ds(s * LANES, LANES)] for s in range(SEG)]
    bvec = [gb_v[1, pl.ds(s * LANES, LANES)] for s in range(SEG)]

    def idx_copy(p, slot):
        return pltpu.make_async_copy(
            ids_hbm.at[:, pl.ds(base + p * (2 * CHUNK), 2 * CHUNK)],
            ids_v.at[slot, pl.ds(0, 5)], isem)

    def issue(slot, off, s):
        for j in range(5):
            for h in range(SPLIT):
                pltpu.async_copy(
                    tabs[j].at[ids_v.at[slot, j, pl.ds(off + h * SUB, SUB)]],
                    bufs[s].at[j, pl.ds(h * SUB, SUB)], gsem[s])

    def drain_gathers(slot, off, s):
        for j in range(5):
            for h in range(SPLIT):
                pltpu.make_async_copy(
                    tabs[j].at[ids_v.at[slot, j, pl.ds(off + h * SUB, SUB)]],
                    bufs[s].at[j, pl.ds(h * SUB, SUB)], gsem[s]).wait()

    def out_copy(g, s):
        return pltpu.make_async_copy(
            bufs[s].at[0], out_hbm.at[pl.ds(base + g * CHUNK, CHUNK)],
            osem[s])

    idx_copy(0, 0).start()
    idx_copy(0, 0).wait()
    issue(0, 0, 0)

    def quad_body(q, carry):
        for u in (0, 1):
          p = 2 * q + u
          pa = u
          pb = 1 - u
          @pl.when(p + 1 < NPAIR)
          def _():
              idx_copy(p + 1, pb).start()

          for s in (0, 1):
            g = 2 * p + s
            nxt = 1 - s
            @pl.when(g + 1 < NCHUNK)
            def _():
                @pl.when(g >= 1)
                def _():
                    out_copy(g - 1, nxt).wait()
                if s == 0:
                    issue(pa, CHUNK, nxt)
                else:
                    idx_copy(p + 1, pb).wait()
                    issue(pb, 0, nxt)

            drain_gathers(pa, s * CHUNK, s)
            b = bufs[s]

            def row(r, rc):
                vs = []
                for seg in range(SEG):
                    sl = pl.ds(seg * LANES, LANES)
                    vs.append(b[0, r, sl] + b[1, r, sl] + b[2, r, sl]
                              + b[3, r, sl] + b[4, r, sl])
                tot = vs[0]
                for seg in range(1, SEG):
                    tot = tot + vs[seg]
                mean = lax.broadcast(jnp.sum(tot) * (1.0 / D), (LANES,))
                xs = [v - mean for v in vs]
                sq = xs[0] * xs[0]
                for seg in range(1, SEG):
                    sq = sq + xs[seg] * xs[seg]
                var = lax.broadcast(jnp.sum(sq) * (1.0 / D) + EPS, (LANES,))
                rstd = _rsqrt(var)
                for seg in range(SEG):
                    sl = pl.ds(seg * LANES, LANES)
                    b[0, r, sl] = xs[seg] * rstd * gvec[seg] + bvec[seg]
                return rc

            lax.fori_loop(0, CHUNK, row, 0, unroll=1)
            out_copy(g, s).start()
        return carry

    lax.fori_loop(0, NPAIR // 2, quad_body, 0, unroll=1)
    out_copy(NCHUNK - 2, 0).wait()
    out_copy(NCHUNK - 1, 1).wait()


@jax.jit
def _poiembed_sc(ids, word_tab, wpos_tab, ppos_tab, grid_tab, cate_tab,
                 ln_gamma, ln_beta):
    mesh = plsc.VectorSubcoreMesh(core_axis_name="c", subcore_axis_name="s")
    f = pl.kernel(
        _body,
        out_type=jax.ShapeDtypeStruct((N, D), jnp.float32),
        mesh=mesh,
        compiler_params=pltpu.CompilerParams(needs_layout_passes=False),
        scratch_types=[
            pltpu.VMEM((2, 8, 2 * CHUNK), jnp.int32),
            pltpu.VMEM((5, CHUNK, D), jnp.float32),
            pltpu.VMEM((5, CHUNK, D), jnp.float32),
            pltpu.VMEM((2, D), jnp.float32),
            pltpu.SemaphoreType.DMA,
            pltpu.SemaphoreType.DMA,
            pltpu.SemaphoreType.DMA,
            pltpu.SemaphoreType.DMA,
            pltpu.SemaphoreType.DMA,
        ],
    )
    return f(ids, word_tab, wpos_tab, ppos_tab, grid_tab, cate_tab,
             ln_gamma, ln_beta)


def kernel(poi_name_token_ids, word_level_pos_ids, poi_level_pos_ids,
           grid_level_pos_ids, poi_cate_ids,
           word_tab, wpos_tab, ppos_tab, grid_tab, cate_tab,
           ln_gamma, ln_beta):
    ids = jnp.stack([
        poi_name_token_ids.reshape(-1),
        word_level_pos_ids.reshape(-1),
        poi_level_pos_ids.reshape(-1),
        grid_level_pos_ids.reshape(-1),
        poi_cate_ids.reshape(-1),
    ], axis=0)
    out = _poiembed_sc(ids, word_tab, wpos_tab, ppos_tab, grid_tab,
                       cate_tab, ln_gamma, ln_beta)
    return out.reshape(B, L, D)
